# Initial kernel scaffold; baseline (speedup 1.0000x reference)
import functools

import jax
import jax.numpy as jnp
from jax import lax
from jax.experimental import pallas as pl
from jax.experimental.pallas import tpu as pltpu
from jax.experimental.pallas import tpu_sc as plsc

N = 1_000_000
D = 16
L = 16
NC, NS = 2, 16
NW = NC * NS

ROWS_PER_TILE = (N + NW - 1) // NW
CHUNK_ROWS = 2608
GROUPS = CHUNK_ROWS // L
CHUNKS = (ROWS_PER_TILE + CHUNK_ROWS - 1) // CHUNK_ROWS
MAX_OFF = N - CHUNK_ROWS

_mesh = plsc.VectorSubcoreMesh(core_axis_name="c", subcore_axis_name="s")

F32_INF = jnp.float32(jnp.inf)
I32_MAX = jnp.int32(2**31 - 1)


@functools.partial(
    pl.kernel,
    out_type=(
        jax.ShapeDtypeStruct((NW, L), jnp.float32),
        jax.ShapeDtypeStruct((NW, L), jnp.int32),
    ),
    mesh=_mesh,
    scratch_types=[
        pltpu.VMEM((2, CHUNK_ROWS, D), jnp.float32),
        pltpu.VMEM((D, L), jnp.float32),
        pltpu.VMEM((L,), jnp.float32),
        pltpu.VMEM((L,), jnp.int32),
        pltpu.SemaphoreType.DMA,
        pltpu.SemaphoreType.DMA,
    ],
)
def _scan_kernel(qb_hbm, keys_hbm, bd_hbm, bi_hbm, buf, qb_v, od_v, oi_v,
                 sem0, sem1):
    wid = lax.axis_index("c") * NS + lax.axis_index("s")
    sems = (sem0, sem1)

    pltpu.sync_copy(qb_hbm, qb_v)
    qb = [qb_v[d] for d in range(D)]
    cols = [jnp.full((L,), d, dtype=jnp.int32) for d in range(D)]
    lane = lax.iota(jnp.int32, L)

    def chunk_off(c):
        return jnp.minimum(wid * ROWS_PER_TILE + c * CHUNK_ROWS, MAX_OFF)

    def start(c):
        s = c % 2
        return pltpu.async_copy(
            keys_hbm.at[pl.ds(chunk_off(c), CHUNK_ROWS)], buf.at[s], sems[s])

    best_d = jnp.full((L,), F32_INF, dtype=jnp.float32)
    best_i = jnp.full((L,), I32_MAX, dtype=jnp.int32)

    copies = [start(0), None]
    for c in range(CHUNKS):
        s = c % 2
        if c + 1 < CHUNKS:
            copies[1 - s] = start(c + 1)
        copies[s].wait()
        off = chunk_off(c)
        cbuf = buf.at[s]

        def gbody(g, carry, cbuf=cbuf, off=off):
            bd, bi, row = carry
            acc = None
            for d in range(D):
                v = plsc.load_gather(cbuf, [row, cols[d]])
                t = v * (v - qb[d])
                acc = t if acc is None else acc + t
            gidx = row + off
            lt = acc < bd
            bd = jnp.where(lt, acc, bd)
            bi = jnp.where(lt, gidx, bi)
            return bd, bi, row + L

        best_d, best_i, _ = lax.fori_loop(
            0, GROUPS, gbody, (best_d, best_i, lane))

    od_v[...] = best_d
    oi_v[...] = best_i
    pltpu.sync_copy(od_v, bd_hbm.at[wid])
    pltpu.sync_copy(oi_v, bi_hbm.at[wid])


@functools.partial(
    pl.kernel,
    out_type=jax.ShapeDtypeStruct((L,), jnp.float32),
    mesh=_mesh,
    scratch_types=[
        pltpu.VMEM((NW, L), jnp.float32),
        pltpu.VMEM((NW, L), jnp.int32),
        pltpu.VMEM((L,), jnp.float32),
        pltpu.VMEM((L,), jnp.int32),
        pltpu.VMEM((L, D), jnp.float32),
        pltpu.VMEM((L,), jnp.float32),
        pltpu.SemaphoreType.DMA,
    ],
)
def _merge_kernel(bd_hbm, bi_hbm, q_hbm, values_hbm, out_hbm,
                  bd_v, bi_v, q_v, idx_v, row_v, out_v, sem):
    wid = lax.axis_index("c") * NS + lax.axis_index("s")

    @pl.when(wid == 0)
    def _():
        pltpu.sync_copy(bd_hbm, bd_v)
        pltpu.sync_copy(bi_hbm, bi_v)
        pltpu.sync_copy(q_hbm, q_v)
        bd = bd_v[0]
        bi = bi_v[0]
        for t in range(1, NW):
            d = bd_v[t]
            i = bi_v[t]
            lt = (d < bd) | ((d == bd) & (i < bi))
            bd = jnp.where(lt, d, bd)
            bi = jnp.where(lt, i, bi)
        m = jnp.min(bd)
        cand = jnp.where(bd == m, bi, I32_MAX)
        win = jnp.min(cand)
        idx_v[...] = jnp.full((L,), win, dtype=jnp.int32)
        pltpu.async_copy(values_hbm.at[idx_v], row_v, sem).wait()
        s = jnp.sum(row_v[0] * q_v[...])
        out_v[...] = jnp.full((L,), s, dtype=jnp.float32)
        pltpu.sync_copy(out_v, out_hbm)


def kernel(query, keys, values):
    qb = jnp.broadcast_to((2.0 * query)[:, None], (D, L))
    bd, bi = _scan_kernel(qb, keys)
    out16 = _merge_kernel(bd, bi, query, values)
    return out16[:1]


# speedup vs baseline: 25.6612x; 25.6612x over previous
import functools

import jax
import jax.numpy as jnp
from jax import lax
from jax.experimental import pallas as pl
from jax.experimental.pallas import tpu as pltpu
from jax.experimental.pallas import tpu_sc as plsc

N = 1_000_000
D = 16
L = 16
NC, NS = 2, 16
NW = NC * NS

TILE_STRIDE = 31232
CHUNK_ROWS = 2560
GROUPS = CHUNK_ROWS // L
CHUNKS = 13
MAX_OFF = 999936 - CHUNK_ROWS
TAIL = 128

_mesh = plsc.VectorSubcoreMesh(
    core_axis_name="c", subcore_axis_name="s", num_cores=NC, num_subcores=NS)

F32_INF = float("inf")
I32_MAX = 2**31 - 1


@functools.partial(
    pl.kernel,
    out_type=(
        jax.ShapeDtypeStruct((NW, L), jnp.float32),
        jax.ShapeDtypeStruct((NW, L), jnp.int32),
    ),
    mesh=_mesh,
    scratch_types=[
        pltpu.VMEM((D, CHUNK_ROWS), jnp.float32),
        pltpu.VMEM((D, CHUNK_ROWS), jnp.float32),
        pltpu.VMEM((D, L), jnp.float32),
        pltpu.VMEM((L,), jnp.float32),
        pltpu.VMEM((L,), jnp.int32),
        pltpu.SemaphoreType.DMA,
        pltpu.SemaphoreType.DMA,
    ],
    compiler_params=pltpu.CompilerParams(needs_layout_passes=False),
)
def _scan_kernel(qb_hbm, keyst_hbm, bd_hbm, bi_hbm, buf0, buf1, qb_v, od_v,
                 oi_v, sem0, sem1):
    wid = lax.axis_index("c") * NS + lax.axis_index("s")
    sems = (sem0, sem1)
    bufs = (buf0, buf1)

    pltpu.sync_copy(qb_hbm, qb_v)
    qb = [qb_v[d] for d in range(D)]
    lane = lax.iota(jnp.int32, L)

    def chunk_off(c):
        return jnp.minimum(wid * TILE_STRIDE + c * CHUNK_ROWS, MAX_OFF)

    def start(c):
        s = c % 2
        return pltpu.async_copy(
            keyst_hbm.at[:, pl.ds(chunk_off(c), CHUNK_ROWS)],
            bufs[s], sems[s])

    best_d = jnp.full((L,), F32_INF, dtype=jnp.float32)
    best_i = jnp.full((L,), I32_MAX, dtype=jnp.int32)

    copies = [start(0), None]
    for c in range(CHUNKS):
        s = c % 2
        if c + 1 < CHUNKS:
            copies[1 - s] = start(c + 1)
        copies[s].wait()
        off = chunk_off(c)
        cbuf = bufs[s]

        def gbody(g, carry, cbuf=cbuf):
            bd, bi, row = carry
            j = g * L
            acc = None
            for d in range(D):
                v = cbuf[d, pl.ds(j, L)]
                t = v * (v - qb[d])
                acc = t if acc is None else acc + t
            lt = acc < bd
            bd = jnp.where(lt, acc, bd)
            bi = jnp.where(lt, row, bi)
            return bd, bi, row + L

        best_d, best_i, _ = lax.fori_loop(
            0, GROUPS, gbody, (best_d, best_i, lane + off))

    od_v[...] = best_d
    oi_v[...] = best_i
    pltpu.sync_copy(od_v, bd_hbm.at[wid])
    pltpu.sync_copy(oi_v, bi_hbm.at[wid])


def _merge_tc_body(bd_ref, bi_ref, q_ref, tail_ref, valuest_hbm, out_ref,
                   blk_v, sem):
    bd = bd_ref[...]
    bi = bi_ref[...]
    m1 = jnp.min(bd)
    win1 = jnp.min(jnp.where(bd == m1, bi, I32_MAX))
    q_col = q_ref[0].reshape(D, 1)
    tail = tail_ref[...]
    td = jnp.sum(tail * (tail - 2.0 * q_col), axis=0, keepdims=True)
    m2 = jnp.min(td)
    jidx = lax.broadcasted_iota(jnp.int32, (1, TAIL), 1) + (N - TAIL)
    win2 = jnp.min(jnp.where(td == m2, jidx, I32_MAX))
    take2 = (m2 < m1) | ((m2 == m1) & (win2 < win1))
    win = jnp.where(take2, win2, win1)
    base = jnp.minimum((win // 128) * 128, N - 128)
    base = pl.multiple_of(base, 128)
    copy = pltpu.make_async_copy(
        valuest_hbm.at[:, pl.ds(base, 128)], blk_v, sem)
    copy.start()
    copy.wait()
    sel = lax.broadcasted_iota(jnp.int32, (1, 128), 1) == (win - base)
    row = jnp.sum(jnp.where(sel, blk_v[...], 0.0), axis=1)
    out_ref[0, 0] = jnp.sum(row * q_ref[0])


_merge_tc = pl.pallas_call(
    _merge_tc_body,
    out_shape=jax.ShapeDtypeStruct((1, 1), jnp.float32),
    in_specs=[
        pl.BlockSpec(memory_space=pltpu.VMEM),
        pl.BlockSpec(memory_space=pltpu.VMEM),
        pl.BlockSpec(memory_space=pltpu.VMEM),
        pl.BlockSpec(memory_space=pltpu.VMEM),
        pl.BlockSpec(memory_space=pltpu.HBM),
    ],
    out_specs=pl.BlockSpec(memory_space=pltpu.SMEM),
    scratch_shapes=[
        pltpu.VMEM((D, 128), jnp.float32),
        pltpu.SemaphoreType.DMA,
    ],
)


def kernel(query, keys, values):
    keys_t = keys.T
    qb = jnp.broadcast_to((2.0 * query)[:, None], (D, L))
    bd, bi = _scan_kernel(qb, keys_t)
    out = _merge_tc(bd, bi, query.reshape(1, D), keys_t[:, N - TAIL:],
                    values.T)
    return out[0]
